# Initial kernel scaffold; baseline (speedup 1.0000x reference)
import jax
import jax.numpy as jnp
from jax.experimental import pallas as pl


def kernel(t_in, y_in, inds_1r, inds_1p, inds_2r, inds_2p, k1, ea1, k2, ea2):
    raise NotImplementedError("write your pallas kernel here")


# speedup vs baseline: 9.9383x; 9.9383x over previous
import functools

import jax
import jax.numpy as jnp
from jax import lax
from jax.experimental import pallas as pl
from jax.experimental.pallas import tpu as pltpu
from jax.experimental.pallas import tpu_sc as plsc

B = 64
N = 16384
R1 = 131072
R2 = 131072

NC = 2
NS = 16
L = 16

GRP = 1024
HALF = 256
DMA = 128
NDMA = HALF // DMA
PER_TILE = R1 // (NC * NS)
NGRP = PER_TILE // GRP
ROWS_PER_TILE = N // NS

_SPLAT_DNUMS = lax.GatherDimensionNumbers(
    offset_dims=(), collapsed_slice_dims=(0,), start_index_map=(0,))


def _splat(vec, j):
    idx = jnp.full((L, 1), j, jnp.int32)
    return lax.gather(vec, idx, _SPLAT_DNUMS, (1,),
                      mode=lax.GatherScatterMode.PROMISE_IN_BOUNDS)


def _sc_reaction(t16, yT, i1r, i1p, i2a, i2b, i2p, k1, ea1, k2, ea2):
    mesh = plsc.VectorSubcoreMesh(core_axis_name="c", subcore_axis_name="s")

    @functools.partial(
        pl.kernel,
        mesh=mesh,
        out_type=jax.ShapeDtypeStruct((NC, N, B), jnp.float32),
        compiler_params=pltpu.CompilerParams(use_tc_tiling_on_sc=False),
        scratch_types=[
            pltpu.VMEM_SHARED((N, B), jnp.float32),
            pltpu.VMEM((L,), jnp.float32),
            pltpu.VMEM((GRP,), jnp.float32),
            pltpu.VMEM((GRP,), jnp.float32),
            pltpu.VMEM((GRP,), jnp.float32),
            pltpu.VMEM((GRP // DMA, DMA), jnp.int32),
            pltpu.VMEM((GRP // DMA, DMA), jnp.int32),
            pltpu.VMEM((GRP // DMA, DMA), jnp.int32),
            pltpu.VMEM((HALF, B), jnp.float32),
            pltpu.VMEM((HALF, B), jnp.float32),
            pltpu.SemaphoreType.DMA,
        ],
    )
    def k(t_hbm, yT_hbm, i1r_hbm, i1p_hbm, i2a_hbm, i2b_hbm, i2p_hbm,
          k1_hbm, ea1_hbm, k2_hbm, ea2_hbm, out_hbm,
          acc, t_v, kbuf, eabuf, rate_v, idxa_v, idxb_v, idxp_v,
          rows_v, rowsb_v, sem):
        c = lax.axis_index("c")
        s = lax.axis_index("s")
        wid = c * NS + s
        base_row = s * ROWS_PER_TILE

        def zrow(i, carry):
            z = jnp.zeros((L,), jnp.float32)
            for q in range(B // L):
                rows_v[i, pl.ds(q * L, L)] = z
            return carry
        lax.fori_loop(0, HALF, zrow, 0)
        for j in range(ROWS_PER_TILE // HALF):
            pltpu.sync_copy(rows_v, acc.at[pl.ds(base_row + j * HALF, HALF)])

        pltpu.sync_copy(t_hbm, t_v)
        plsc.subcore_barrier()

        nt = -t_v[...]

        def compute_rates(kh, eah, g):
            pltpu.sync_copy(kh.at[pl.ds(g, GRP)], kbuf)
            pltpu.sync_copy(eah.at[pl.ds(g, GRP)], eabuf)

            def rbody(i, carry):
                o = i * L
                kv = kbuf[pl.ds(o, L)]
                ev = eabuf[pl.ds(o, L)]
                rate_v[pl.ds(o, L)] = kv * jnp.exp(ev * nt)
                return carry
            lax.fori_loop(0, GRP // L, rbody, 0)

        def gather(idx_ref, half, dst_ref):
            cps = [
                pltpu.async_copy(yT_hbm.at[idx_ref.at[half * NDMA + j]],
                                 dst_ref.at[pl.ds(j * DMA, DMA)], sem)
                for j in range(NDMA)
            ]
            for cp in cps:
                cp.wait()

        def scatter_add(src_ref, idx_ref, half):
            for j in range(NDMA):
                pltpu.sync_copy(src_ref.at[pl.ds(j * DMA, DMA)],
                                acc.at[idx_ref.at[half * NDMA + j]], add=True)

        def grp1(gi, carry):
            g = pl.multiple_of(wid * PER_TILE + gi * GRP, GRP)
            grow = pl.multiple_of(g // DMA, GRP // DMA)
            compute_rates(k1_hbm, ea1_hbm, g)
            pltpu.sync_copy(i1r_hbm.at[pl.ds(grow, GRP // DMA)], idxa_v)
            pltpu.sync_copy(i1p_hbm.at[pl.ds(grow, GRP // DMA)], idxp_v)
            for half in range(GRP // HALF):
                gather(idxa_v, half, rows_v)
                rbase = half * HALF

                def sbody(i, carry2):
                    rv = rate_v[pl.ds(pl.multiple_of(rbase + i * L, L), L)]
                    for j in range(L):
                        r = i * L + j
                        rs = _splat(rv, j)
                        for q in range(B // L):
                            sl = pl.ds(q * L, L)
                            rows_v[r, sl] = rows_v[r, sl] * rs
                    return carry2
                lax.fori_loop(0, HALF // L, sbody, 0)
                scatter_add(rows_v, idxp_v, half)
            return carry
        lax.fori_loop(0, NGRP, grp1, 0)

        def grp2(gi, carry):
            g = pl.multiple_of(wid * PER_TILE + gi * GRP, GRP)
            grow = pl.multiple_of(g // DMA, GRP // DMA)
            compute_rates(k2_hbm, ea2_hbm, g)
            pltpu.sync_copy(i2a_hbm.at[pl.ds(grow, GRP // DMA)], idxa_v)
            pltpu.sync_copy(i2b_hbm.at[pl.ds(grow, GRP // DMA)], idxb_v)
            pltpu.sync_copy(i2p_hbm.at[pl.ds(grow, GRP // DMA)], idxp_v)
            for half in range(GRP // HALF):
                gather(idxa_v, half, rows_v)
                gather(idxb_v, half, rowsb_v)
                rbase = half * HALF

                def sbody(i, carry2):
                    rv = rate_v[pl.ds(pl.multiple_of(rbase + i * L, L), L)]
                    for j in range(L):
                        r = i * L + j
                        rs = _splat(rv, j)
                        for q in range(B // L):
                            sl = pl.ds(q * L, L)
                            rows_v[r, sl] = rows_v[r, sl] * rowsb_v[r, sl] * rs
                    return carry2
                lax.fori_loop(0, HALF // L, sbody, 0)
                scatter_add(rows_v, idxp_v, half)
            return carry
        lax.fori_loop(0, NGRP, grp2, 0)

        plsc.subcore_barrier()
        pltpu.sync_copy(acc.at[pl.ds(base_row, ROWS_PER_TILE)],
                        out_hbm.at[c].at[pl.ds(base_row, ROWS_PER_TILE)])

    return k(t16, yT, i1r, i1p, i2a, i2b, i2p, k1, ea1, k2, ea2)


def kernel(t_in, y_in, inds_1r, inds_1p, inds_2r, inds_2p, k1, ea1, k2, ea2):
    t16 = jnp.broadcast_to(t_in.astype(jnp.float32), (L,))
    yT = y_in.T
    i1r = inds_1r.astype(jnp.int32).reshape(R1 // DMA, DMA)
    i1p = inds_1p.astype(jnp.int32).reshape(R1 // DMA, DMA)
    i2a = inds_2r[:, 0].astype(jnp.int32).reshape(R2 // DMA, DMA)
    i2b = inds_2r[:, 1].astype(jnp.int32).reshape(R2 // DMA, DMA)
    i2p = inds_2p.astype(jnp.int32).reshape(R2 // DMA, DMA)
    partials = _sc_reaction(t16, yT, i1r, i1p, i2a, i2b, i2p,
                            k1, ea1, k2, ea2)
    return (partials[0] + partials[1]).T
